# baseline (speedup 1.0000x reference)
import functools

import jax
import jax.numpy as jnp
from jax import lax
from jax.experimental import pallas as pl
from jax.experimental.pallas import tpu as pltpu
from jax.experimental.pallas import tpu_sc as plsc

_N = 100000

_NW = 32
_ECHUNK = 1024
_DCHUNK = 640


def _ew_body(m_ref, l_ref, r_ref, sq_ref, tril_ref):
    sq_ref[...] = m_ref[...] * m_ref[...]
    tril_ref[...] = -(l_ref[...] * r_ref[...])


def _elementwise(maps, left, right):
    two_e, d = maps.shape
    e = left.shape[0]
    rows = 25000
    m2 = maps.reshape(rows, two_e * d // rows)
    l2 = left.reshape(rows, e * d // rows)
    r2 = right.reshape(rows, e * d // rows)
    blk = 1000
    sq, tril = pl.pallas_call(
        _ew_body,
        grid=(rows // blk,),
        in_specs=[
            pl.BlockSpec((blk, m2.shape[1]), lambda i: (i, 0)),
            pl.BlockSpec((blk, l2.shape[1]), lambda i: (i, 0)),
            pl.BlockSpec((blk, l2.shape[1]), lambda i: (i, 0)),
        ],
        out_specs=[
            pl.BlockSpec((blk, m2.shape[1]), lambda i: (i, 0)),
            pl.BlockSpec((blk, l2.shape[1]), lambda i: (i, 0)),
        ],
        out_shape=[
            jax.ShapeDtypeStruct(m2.shape, jnp.float32),
            jax.ShapeDtypeStruct(l2.shape, jnp.float32),
        ],
    )(m2, l2, r2)
    return sq.reshape(two_e, d), tril.reshape(e, d)


def _make_assemble(E, D, total):
    n_echunk = E // _ECHUNK
    e_iters = -(-n_echunk // _NW)
    nd = _N * D
    n_dchunk = nd // _DCHUNK
    d_iters = -(-n_dchunk // _NW)
    erows = _ECHUNK * D // 128

    mesh = plsc.VectorSubcoreMesh(core_axis_name="c", subcore_axis_name="s")

    @functools.partial(
        pl.kernel,
        out_type=[
            jax.ShapeDtypeStruct((2 * total,), jnp.int32),
            jax.ShapeDtypeStruct((total,), jnp.float32),
        ],
        mesh=mesh,
        scratch_types=[
            pltpu.VMEM((_ECHUNK,), jnp.int32),
            pltpu.VMEM((_ECHUNK,), jnp.int32),
            pltpu.VMEM((_ECHUNK,), jnp.int32),
            pltpu.VMEM((_ECHUNK,), jnp.int32),
            pltpu.VMEM((_ECHUNK * D,), jnp.float32),
            pltpu.VMEM((erows, 128), jnp.int32),
            pltpu.VMEM((erows, 128), jnp.int32),
            pltpu.VMEM((erows, 128), jnp.int32),
            pltpu.VMEM((erows, 128), jnp.int32),
            pltpu.VMEM((_DCHUNK // 128, 128), jnp.int32),
            pltpu.VMEM((_DCHUNK // 128, 128), jnp.int32),
            pltpu.VMEM((_DCHUNK,), jnp.float32),
            pltpu.VMEM((_DCHUNK // 128, 128), jnp.int32),
            pltpu.SemaphoreType.DMA,
        ],
        compiler_params=pltpu.CompilerParams(needs_layout_passes=False),
    )
    def assemble(base_hbm, stride_hbm, su_hbm, sw_hbm, val_hbm,
                 dslot_hbm, dval_hbm, out01_hbm, outw_hbm,
                 base_v, stride_v, su_v, sw_v, val_v,
                 slot0_b, slot1_b, o0_b, o1_b,
                 dslot_v, dslot2_v, dval_v, dp_b, sem):
        wid = lax.axis_index("s") * 2 + lax.axis_index("c")
        iota = lax.iota(jnp.int32, 16)
        rep = lax.shift_right_logical(iota, 2)
        ch = lax.bitwise_and(iota, 3)

        def edge_chunk(k, _):
            c = wid + _NW * k

            @pl.when(c < n_echunk)
            def _():
                off = c * _ECHUNK
                pltpu.sync_copy(base_hbm.at[pl.ds(off, _ECHUNK)], base_v)
                pltpu.sync_copy(stride_hbm.at[pl.ds(off, _ECHUNK)], stride_v)
                pltpu.sync_copy(su_hbm.at[pl.ds(off, _ECHUNK)], su_v)
                pltpu.sync_copy(sw_hbm.at[pl.ds(off, _ECHUNK)], sw_v)
                pltpu.sync_copy(val_hbm.at[pl.ds(off * D, _ECHUNK * D)], val_v)

                def expand(r, _):
                    for q in range(8):
                        e16 = 32 * r + 4 * q + rep
                        b16 = plsc.load_gather(base_v, [e16])
                        s16 = plsc.load_gather(stride_v, [e16])
                        u16 = plsc.load_gather(su_v, [e16])
                        w16 = plsc.load_gather(sw_v, [e16])
                        slot = b16 + ch * s16
                        cs = pl.ds(16 * q, 16)
                        slot0_b[r, cs] = slot
                        slot1_b[r, cs] = slot + total
                        o0_b[r, cs] = u16 * D + ch
                        o1_b[r, cs] = w16 * D + ch
                    return _

                lax.fori_loop(0, erows, expand, None)

                def fire(r, _):
                    pltpu.async_copy(
                        o0_b.at[r], out01_hbm.at[slot0_b.at[r]], sem)
                    pltpu.async_copy(
                        o1_b.at[r], out01_hbm.at[slot1_b.at[r]], sem)
                    pltpu.async_copy(
                        val_v.at[pl.ds(128 * r, 128)],
                        outw_hbm.at[slot0_b.at[r]], sem)
                    return _

                def drain(r, _):
                    pltpu.make_async_copy(
                        o0_b.at[r], out01_hbm.at[slot0_b.at[r]], sem).wait()
                    pltpu.make_async_copy(
                        o1_b.at[r], out01_hbm.at[slot1_b.at[r]], sem).wait()
                    pltpu.make_async_copy(
                        val_v.at[pl.ds(128 * r, 128)],
                        outw_hbm.at[slot0_b.at[r]], sem).wait()
                    return _

                lax.fori_loop(0, erows, fire, None)
                lax.fori_loop(0, erows, drain, None)
            return _

        lax.fori_loop(0, e_iters, edge_chunk, None)

        def diag_chunk(k, _):
            c = wid + _NW * k

            @pl.when(c < n_dchunk)
            def _():
                pltpu.sync_copy(dslot_hbm.at[c], dslot_v)
                pltpu.sync_copy(dval_hbm.at[pl.ds(c * _DCHUNK, _DCHUNK)],
                                dval_v)
                for r in range(_DCHUNK // 128):
                    for q in range(8):
                        cs = pl.ds(16 * q, 16)
                        p16 = c * _DCHUNK + 128 * r + 16 * q + iota
                        dp_b[r, cs] = p16
                        dslot2_v[r, cs] = dslot_v[r, cs] + total
                def dfire(r, _):
                    pltpu.async_copy(
                        dp_b.at[r], out01_hbm.at[dslot_v.at[r]], sem)
                    pltpu.async_copy(
                        dp_b.at[r], out01_hbm.at[dslot2_v.at[r]], sem)
                    pltpu.async_copy(
                        dval_v.at[pl.ds(128 * r, 128)],
                        outw_hbm.at[dslot_v.at[r]], sem)
                    return _

                def ddrain(r, _):
                    pltpu.make_async_copy(
                        dp_b.at[r], out01_hbm.at[dslot_v.at[r]], sem).wait()
                    pltpu.make_async_copy(
                        dp_b.at[r], out01_hbm.at[dslot2_v.at[r]], sem).wait()
                    pltpu.make_async_copy(
                        dval_v.at[pl.ds(128 * r, 128)],
                        outw_hbm.at[dslot_v.at[r]], sem).wait()
                    return _

                lax.fori_loop(0, _DCHUNK // 128, dfire, None)
                lax.fori_loop(0, _DCHUNK // 128, ddrain, None)
            return _

        lax.fori_loop(0, d_iters, diag_chunk, None)

    return assemble


def kernel(maps, edge_index):
    E = edge_index.shape[1]
    e_half = E // 2
    D = maps.shape[1]
    u_all = edge_index[0]
    w_all = edge_index[1]

    maps_sq, tril_maps = _elementwise(maps, maps[:e_half], maps[e_half:])

    diag = jax.ops.segment_sum(maps_sq, u_all, num_segments=_N)
    dsi = jnp.power(diag + 1.0, -0.5)
    tril_norm = dsi[u_all[:e_half]] * tril_maps * dsi[w_all[:e_half]]
    diag_norm = dsi * diag * dsi

    ones_e = jnp.ones((E,), jnp.int32)
    deg = jax.ops.segment_sum(ones_e, u_all, num_segments=_N)
    dtril = jax.ops.segment_sum(ones_e[:e_half], u_all[:e_half], num_segments=_N)
    zero1 = jnp.zeros((1,), jnp.int32)
    row_start_edges = jnp.concatenate([zero1, jnp.cumsum(deg)[:-1]])
    node_start = jnp.concatenate([zero1, D * jnp.cumsum(deg + 1)[:-1]])

    perm = jnp.lexsort((w_all, u_all))
    su = u_all[perm]
    sw = w_all[perm]
    sval = jnp.concatenate([tril_norm, tril_norm], axis=0)[perm]

    s = jnp.arange(E, dtype=jnp.int32)
    base = node_start[su] + (s - row_start_edges[su]) + (sw > su).astype(jnp.int32)
    stride = deg[su] + 1

    i = jnp.arange(D, dtype=jnp.int32)
    vslots = (node_start[:, None] + i[None, :] * (deg + 1)[:, None]
              + dtril[:, None]).reshape(-1)

    total = E * D + _N * D
    assemble = _make_assemble(E, D, total)
    out01, outw = assemble(
        base, stride, su, sw, sval.reshape(-1),
        vslots.reshape(-1, _DCHUNK // 128, 128), diag_norm.reshape(-1))
    return (out01.reshape(2, total), outw), tril_maps


# speedup vs baseline: 1.4938x; 1.4938x over previous
import functools

import jax
import jax.numpy as jnp
from jax import lax
from jax.experimental import pallas as pl
from jax.experimental.pallas import tpu as pltpu
from jax.experimental.pallas import tpu_sc as plsc

_N = 100000

_NW = 32
_ECHUNK = 1024
_DCHUNK = 640


def _ew_body(m_ref, l_ref, r_ref, sq_ref, tril_ref):
    sq_ref[...] = m_ref[...] * m_ref[...]
    tril_ref[...] = -(l_ref[...] * r_ref[...])


def _elementwise(maps, left, right):
    two_e, d = maps.shape
    e = left.shape[0]
    rows = 25000
    m2 = maps.reshape(rows, two_e * d // rows)
    l2 = left.reshape(rows, e * d // rows)
    r2 = right.reshape(rows, e * d // rows)
    blk = 1000
    sq, tril = pl.pallas_call(
        _ew_body,
        grid=(rows // blk,),
        in_specs=[
            pl.BlockSpec((blk, m2.shape[1]), lambda i: (i, 0)),
            pl.BlockSpec((blk, l2.shape[1]), lambda i: (i, 0)),
            pl.BlockSpec((blk, l2.shape[1]), lambda i: (i, 0)),
        ],
        out_specs=[
            pl.BlockSpec((blk, m2.shape[1]), lambda i: (i, 0)),
            pl.BlockSpec((blk, l2.shape[1]), lambda i: (i, 0)),
        ],
        out_shape=[
            jax.ShapeDtypeStruct(m2.shape, jnp.float32),
            jax.ShapeDtypeStruct(l2.shape, jnp.float32),
        ],
    )(m2, l2, r2)
    return sq.reshape(two_e, d), tril.reshape(e, d)


def _make_assemble(E, D, total):
    n_echunk = E // _ECHUNK
    e_iters = -(-n_echunk // _NW)
    nd = _N * D
    n_dchunk = nd // _DCHUNK
    d_iters = -(-n_dchunk // _NW)
    erows = _ECHUNK * D // 128

    mesh = plsc.VectorSubcoreMesh(core_axis_name="c", subcore_axis_name="s")

    @functools.partial(
        pl.kernel,
        out_type=jax.ShapeDtypeStruct((total, 8), jnp.int32),
        mesh=mesh,
        scratch_types=[
            pltpu.VMEM((_ECHUNK,), jnp.int32),
            pltpu.VMEM((_ECHUNK,), jnp.int32),
            pltpu.VMEM((_ECHUNK,), jnp.int32),
            pltpu.VMEM((_ECHUNK,), jnp.int32),
            pltpu.VMEM((_ECHUNK * D,), jnp.float32),
            pltpu.VMEM((erows, 128), jnp.int32),
            pltpu.VMEM((erows, 128, 8), jnp.int32),
            pltpu.VMEM((_DCHUNK // 128, 128), jnp.int32),
            pltpu.VMEM((_DCHUNK,), jnp.float32),
            pltpu.VMEM((_DCHUNK // 128, 128, 8), jnp.int32),
            pltpu.SemaphoreType.DMA,
        ],
        compiler_params=pltpu.CompilerParams(needs_layout_passes=False, use_tc_tiling_on_sc=False),
    )
    def assemble(base_hbm, stride_hbm, su_hbm, sw_hbm, val_hbm,
                 dslot_hbm, dval_hbm, outp_hbm,
                 base_v, stride_v, su_v, sw_v, val_v,
                 slot_b, pk_b, dslot_v, dval_v, dpk_b, sem):
        wid = lax.axis_index("s") * 2 + lax.axis_index("c")
        iota = lax.iota(jnp.int32, 16)
        rep = lax.shift_right_logical(iota, 2)
        ch = lax.bitwise_and(iota, 3)
        zero16 = iota * 0

        def edge_chunk(k, _):
            c = wid + _NW * k

            @pl.when(c < n_echunk)
            def _():
                off = c * _ECHUNK
                pltpu.sync_copy(base_hbm.at[pl.ds(off, _ECHUNK)], base_v)
                pltpu.sync_copy(stride_hbm.at[pl.ds(off, _ECHUNK)], stride_v)
                pltpu.sync_copy(su_hbm.at[pl.ds(off, _ECHUNK)], su_v)
                pltpu.sync_copy(sw_hbm.at[pl.ds(off, _ECHUNK)], sw_v)
                pltpu.sync_copy(val_hbm.at[pl.ds(off * D, _ECHUNK * D)], val_v)

                def expand(r, _):
                    rv = zero16 + r
                    for q in range(8):
                        e16 = 32 * r + 4 * q + rep
                        b16 = plsc.load_gather(base_v, [e16])
                        s16 = plsc.load_gather(stride_v, [e16])
                        u16 = plsc.load_gather(su_v, [e16])
                        w16 = plsc.load_gather(sw_v, [e16])
                        wv = val_v[pl.ds(128 * r + 16 * q, 16)]
                        slot_b[r, pl.ds(16 * q, 16)] = b16 + ch * s16
                        k16 = 16 * q + iota
                        plsc.store_scatter(
                            pk_b, [rv, k16, zero16], u16 * D + ch)
                        plsc.store_scatter(
                            pk_b, [rv, k16, zero16 + 1], w16 * D + ch)
                        plsc.store_scatter(
                            pk_b, [rv, k16, zero16 + 2],
                            plsc.bitcast(wv, jnp.int32))
                    return _

                lax.fori_loop(0, erows, expand, None)

                def fire(r, _):
                    pltpu.async_copy(
                        pk_b.at[r], outp_hbm.at[slot_b.at[r]], sem)
                    return _

                def drain(r, _):
                    pltpu.make_async_copy(
                        pk_b.at[r], outp_hbm.at[slot_b.at[r]], sem).wait()
                    return _

                lax.fori_loop(0, erows, fire, None)
                lax.fori_loop(0, erows, drain, None)
            return _

        lax.fori_loop(0, e_iters, edge_chunk, None)

        def diag_chunk(k, _):
            c = wid + _NW * k

            @pl.when(c < n_dchunk)
            def _():
                pltpu.sync_copy(dslot_hbm.at[c], dslot_v)
                pltpu.sync_copy(dval_hbm.at[pl.ds(c * _DCHUNK, _DCHUNK)],
                                dval_v)
                for r in range(_DCHUNK // 128):
                    rv = zero16 + r
                    for q in range(8):
                        p16 = c * _DCHUNK + 128 * r + 16 * q + iota
                        dwv = dval_v[pl.ds(128 * r + 16 * q, 16)]
                        k16 = 16 * q + iota
                        plsc.store_scatter(dpk_b, [rv, k16, zero16], p16)
                        plsc.store_scatter(dpk_b, [rv, k16, zero16 + 1], p16)
                        plsc.store_scatter(
                            dpk_b, [rv, k16, zero16 + 2],
                            plsc.bitcast(dwv, jnp.int32))

                def dfire(r, _):
                    pltpu.async_copy(
                        dpk_b.at[r], outp_hbm.at[dslot_v.at[r]], sem)
                    return _

                def ddrain(r, _):
                    pltpu.make_async_copy(
                        dpk_b.at[r], outp_hbm.at[dslot_v.at[r]], sem).wait()
                    return _

                lax.fori_loop(0, _DCHUNK // 128, dfire, None)
                lax.fori_loop(0, _DCHUNK // 128, ddrain, None)
            return _

        lax.fori_loop(0, d_iters, diag_chunk, None)

    return assemble


def kernel(maps, edge_index):
    E = edge_index.shape[1]
    e_half = E // 2
    D = maps.shape[1]
    u_all = edge_index[0]
    w_all = edge_index[1]

    maps_sq, tril_maps = _elementwise(maps, maps[:e_half], maps[e_half:])

    diag = jax.ops.segment_sum(maps_sq, u_all, num_segments=_N)
    dsi = jnp.power(diag + 1.0, -0.5)
    tril_norm = dsi[u_all[:e_half]] * tril_maps * dsi[w_all[:e_half]]
    diag_norm = dsi * diag * dsi

    ones_e = jnp.ones((E,), jnp.int32)
    deg = jax.ops.segment_sum(ones_e, u_all, num_segments=_N)
    dtril = jax.ops.segment_sum(ones_e[:e_half], u_all[:e_half], num_segments=_N)
    zero1 = jnp.zeros((1,), jnp.int32)
    row_start_edges = jnp.concatenate([zero1, jnp.cumsum(deg)[:-1]])
    node_start = jnp.concatenate([zero1, D * jnp.cumsum(deg + 1)[:-1]])

    perm = jnp.lexsort((w_all, u_all))
    su = u_all[perm]
    sw = w_all[perm]
    sval = jnp.concatenate([tril_norm, tril_norm], axis=0)[perm]

    s = jnp.arange(E, dtype=jnp.int32)
    base = node_start[su] + (s - row_start_edges[su]) + (sw > su).astype(jnp.int32)
    stride = deg[su] + 1

    i = jnp.arange(D, dtype=jnp.int32)
    vslots = (node_start[:, None] + i[None, :] * (deg + 1)[:, None]
              + dtril[:, None]).reshape(-1)

    total = E * D + _N * D
    assemble = _make_assemble(E, D, total)
    outp = assemble(
        base, stride, su, sw, sval.reshape(-1),
        vslots.reshape(-1, _DCHUNK // 128, 128), diag_norm.reshape(-1))
    lap_index = jnp.stack([outp[:, 0], outp[:, 1]])
    weights = lax.bitcast_convert_type(outp[:, 2], jnp.float32)
    return (lap_index, weights), tril_maps
